# baseline (speedup 1.0000x reference)
import functools

import jax
import jax.numpy as jnp
from jax import lax
from jax.experimental import pallas as pl
from jax.experimental.pallas import tpu as pltpu
from jax.experimental.pallas import tpu_sc as plsc

NC, NS = 2, 16
NW = NC * NS
D = 128
E = 200000
PW = 6272
EP = NW * PW
CH = 8192
CSH = 13
N_NODE = 50000
NCHUNK = 7
XP = NCHUNK * CH
CAPP = 7296
GRP = 1024
N_TEST = 1000
TP = 1024
MMB = 1000
NE = 50176
PWE = NE // NW
BBE = 392


def _mesh():
    return plsc.VectorSubcoreMesh(core_axis_name="c", subcore_axis_name="s",
                                  num_cores=NC, num_subcores=NS)


def _sc_params():
    return pltpu.CompilerParams(needs_layout_passes=False)


def _wid():
    return lax.axis_index("s") * NC + lax.axis_index("c")


_GD = lax.GatherDimensionNumbers(offset_dims=(), collapsed_slice_dims=(0,),
                                 start_index_map=(0,))


def _lane_bcast(v16, e):
    idx = jnp.full((16, 1), e, jnp.int32)
    return lax.gather(v16, idx, _GD, slice_sizes=(1,),
                      mode=lax.GatherScatterMode.PROMISE_IN_BOUNDS)


def _mm_body(x_ref, w_ref, b_ref, o_ref, *, in_relu, out_relu):
    x = x_ref[...]
    if in_relu:
        x = jnp.maximum(x, 0.0)
    y = jnp.dot(x, w_ref[...], preferred_element_type=jnp.float32) + b_ref[...]
    if out_relu:
        y = jnp.maximum(y, 0.0)
    o_ref[...] = y


def _mm(x, W, b, in_relu, out_relu):
    nblk = N_NODE // MMB
    return pl.pallas_call(
        functools.partial(_mm_body, in_relu=in_relu, out_relu=out_relu),
        grid=(nblk,),
        in_specs=[pl.BlockSpec((MMB, D), lambda i: (i, 0)),
                  pl.BlockSpec((D, D), lambda i: (0, 0)),
                  pl.BlockSpec((1, D), lambda i: (0, 0))],
        out_specs=pl.BlockSpec((MMB, D), lambda i: (i, 0)),
        out_shape=jax.ShapeDtypeStruct((N_NODE, D), jnp.float32),
    )(x, W, b.reshape(1, D))


def _enc_body(g_ref, f_ref, w_ref, b_ref, o_ref):
    o_ref[...] = (g_ref[...]
                  + jnp.dot(f_ref[...], w_ref[...],
                            preferred_element_type=jnp.float32)
                  + b_ref[...])


def _enc(g, feat, W, b):
    nblk = N_NODE // MMB
    kdim = feat.shape[1]
    return pl.pallas_call(
        _enc_body,
        grid=(nblk,),
        in_specs=[pl.BlockSpec((MMB, D), lambda i: (i, 0)),
                  pl.BlockSpec((MMB, kdim), lambda i: (i, 0)),
                  pl.BlockSpec((kdim, D), lambda i: (0, 0)),
                  pl.BlockSpec((1, D), lambda i: (0, 0))],
        out_specs=pl.BlockSpec((MMB, D), lambda i: (i, 0)),
        out_shape=jax.ShapeDtypeStruct((N_NODE, D), jnp.float32),
    )(g, feat, W, b.reshape(1, D))


def _dec_body(a0_ref, a1_ref, w_ref, b_ref, z_ref, p_ref):
    t = jnp.maximum(a0_ref[...] + a1_ref[...], 0.0)
    z = jnp.dot(t, w_ref[...], preferred_element_type=jnp.float32) + b_ref[...]
    z_ref[...] = z
    m = jnp.max(z, axis=1, keepdims=True)
    ez = jnp.exp(z - m)
    p_ref[...] = ez / jnp.sum(ez, axis=1, keepdims=True)


def _dec(a0, a1, Wdp, bdp):
    return pl.pallas_call(
        _dec_body,
        out_shape=(jax.ShapeDtypeStruct((TP, D), jnp.float32),
                   jax.ShapeDtypeStruct((TP, D), jnp.float32)),
    )(a0, a1, Wdp, bdp.reshape(1, D))


def _emb_gather2(tab0, idx0, tab1, idx1):
    @functools.partial(
        pl.kernel,
        out_type=(jax.ShapeDtypeStruct((NE, D), jnp.float32),
                  jax.ShapeDtypeStruct((NE, D), jnp.float32)),
        mesh=_mesh(),
        compiler_params=_sc_params(),
        scratch_types=[pltpu.VMEM((PWE,), jnp.int32),
                       pltpu.VMEM((BBE, D), jnp.float32),
                       pltpu.SemaphoreType.DMA],
    )
    def k(tab0_h, idx0_h, tab1_h, idx1_h, o0_h, o1_h, idxv, rowsv, sem):
        base = _wid() * PWE
        for tab_h, idx_h, o_h in ((tab0_h, idx0_h, o0_h),
                                  (tab1_h, idx1_h, o1_h)):
            pltpu.sync_copy(idx_h.at[pl.ds(base, PWE)], idxv)

            def body(bi, _, tab_h=tab_h, o_h=o_h):
                off = bi * BBE
                pltpu.async_copy(tab_h.at[idxv.at[pl.ds(off, BBE)]],
                                 rowsv, sem).wait()
                pltpu.sync_copy(rowsv, o_h.at[pl.ds(base + off, BBE)])
                return 0

            lax.fori_loop(0, PWE // BBE, body, 0)

    return k(tab0, idx0, tab1, idx1)


def _bucket(es, w):
    @functools.partial(
        pl.kernel,
        out_type=(jax.ShapeDtypeStruct((NW * NCHUNK * CAPP,), jnp.int32),
                  jax.ShapeDtypeStruct((NW * NCHUNK * CAPP,), jnp.int32),
                  jax.ShapeDtypeStruct((NW * NCHUNK * CAPP,), jnp.float32),
                  jax.ShapeDtypeStruct((NW * 16,), jnp.int32)),
        mesh=_mesh(),
        compiler_params=_sc_params(),
        scratch_types=[pltpu.VMEM((PW,), jnp.int32),
                       pltpu.VMEM((PW,), jnp.int32),
                       pltpu.VMEM((PW,), jnp.float32),
                       pltpu.VMEM((CAPP + 16,), jnp.int32),
                       pltpu.VMEM((CAPP + 16,), jnp.int32),
                       pltpu.VMEM((CAPP + 16,), jnp.float32),
                       pltpu.VMEM((16,), jnp.int32)],
    )
    def k(src_h, tgt_h, w_h, bs_h, bt_h, bw_h, cnt_h, siv, tiv, wiv, csv,
          ctv, cwv, cntv):
        wid = _wid()
        base = wid * PW
        pltpu.sync_copy(src_h.at[pl.ds(base, PW)], siv)
        pltpu.sync_copy(tgt_h.at[pl.ds(base, PW)], tiv)
        pltpu.sync_copy(w_h.at[pl.ds(base, PW)], wiv)
        lane = lax.broadcasted_iota(jnp.int32, (16,), 0)
        z16i = jnp.zeros((16,), jnp.int32)
        z16f = jnp.zeros((16,), jnp.float32)
        cv = jnp.zeros((16,), jnp.int32)
        for c in range(NCHUNK):
            def cb(i, pos, c=c):
                off = i * 16
                s16 = siv[pl.ds(off, 16)]
                t16 = tiv[pl.ds(off, 16)]
                w16 = wiv[pl.ds(off, 16)]
                m = (t16 >> CSH) == c
                mi = m.astype(jnp.int32)
                excl = plsc.cumsum(mi) - mi
                idx = jnp.where(m, pos + excl, CAPP + lane)
                plsc.store_scatter(csv, [idx], s16)
                plsc.store_scatter(ctv, [idx], t16 & (CH - 1))
                plsc.store_scatter(cwv, [idx], w16)
                return pos + jnp.sum(mi)

            n = lax.fori_loop(0, PW // 16, cb, jnp.int32(0))

            def zb(kk, _, ):
                o = n + kk * 16
                csv[pl.ds(o, 16)] = z16i
                ctv[pl.ds(o, 16)] = z16i
                cwv[pl.ds(o, 16)] = z16f
                return 0

            lax.fori_loop(0, GRP // 16, zb, 0)
            bko = (wid * NCHUNK + c) * CAPP
            pltpu.sync_copy(csv.at[pl.ds(0, CAPP)], bs_h.at[pl.ds(bko, CAPP)])
            pltpu.sync_copy(ctv.at[pl.ds(0, CAPP)], bt_h.at[pl.ds(bko, CAPP)])
            pltpu.sync_copy(cwv.at[pl.ds(0, CAPP)], bw_h.at[pl.ds(bko, CAPP)])
            cv = jnp.where(lane == c, n, cv)
        cntv[...] = cv
        pltpu.sync_copy(cntv, cnt_h.at[pl.ds(wid * 16, 16)])

    return k(es[0], es[1], w)


def _scale_scatter_batch(msg_h, acc, sgv, tgv, wgv, rowsv, sem, b):
    pltpu.async_copy(msg_h.at[sgv.at[pl.ds(b * 128, 128)]], rowsv, sem).wait()

    def sb(s, _):
        eo = b * 128 + s * 16
        w16 = wgv[pl.ds(eo, 16)]
        i16 = tgv[pl.ds(eo, 16)]
        for e in range(16):
            r = s * 16 + e
            wb = _lane_bcast(w16, e)
            for f in range(8):
                rowsv[r, pl.ds(f * 16, 16)] = rowsv[r, pl.ds(f * 16, 16)] * wb
        pltpu.sync_copy(rowsv.at[pl.ds(s * 16, 16)], acc.at[i16], add=True)
        return 0

    lax.fori_loop(0, 8, sb, 0)


def _edge_pass(msg, bs, bt, bw, cnt, zeros64):
    @functools.partial(
        pl.kernel,
        out_type=jax.ShapeDtypeStruct((XP, D), jnp.float32),
        mesh=_mesh(),
        compiler_params=_sc_params(),
        scratch_types=[pltpu.VMEM_SHARED((CH, D), jnp.float32),
                       pltpu.VMEM((GRP,), jnp.int32),
                       pltpu.VMEM((GRP,), jnp.int32),
                       pltpu.VMEM((GRP,), jnp.float32),
                       pltpu.VMEM((128, D), jnp.float32),
                       pltpu.VMEM((16,), jnp.int32),
                       pltpu.VMEM((64, D), jnp.float32),
                       pltpu.SemaphoreType.DMA],
    )
    def k(msg_h, bs_h, bt_h, bw_h, cnt_h, z_h, out_h, acc, sgv, tgv, wgv,
          rowsv, cntv, zbuf, sem):
        cid = lax.axis_index("c")
        sid = lax.axis_index("s")
        pltpu.sync_copy(z_h, zbuf)
        lane = lax.broadcasted_iota(jnp.int32, (16,), 0)

        def chunk_body(c, _):
            own = (c & 1) == cid

            @pl.when(own)
            def _zero():
                def zb(z, _2):
                    pltpu.sync_copy(zbuf,
                                    acc.at[pl.ds(sid * 512 + z * 64, 64)])
                    return 0
                lax.fori_loop(0, 8, zb, 0)

            plsc.subcore_barrier()

            @pl.when(own)
            def _proc():
                def tb(j, _2):
                    t = sid * 2 + j
                    pltpu.sync_copy(cnt_h.at[pl.ds(t * 16, 16)], cntv)
                    n = jnp.sum(jnp.where(lane == c, cntv[...], 0))
                    ng = (n + (GRP - 1)) >> 10

                    def gb(g, _3):
                        o = (t * NCHUNK + c) * CAPP + g * GRP
                        pltpu.sync_copy(bs_h.at[pl.ds(o, GRP)], sgv)
                        pltpu.sync_copy(bt_h.at[pl.ds(o, GRP)], tgv)
                        pltpu.sync_copy(bw_h.at[pl.ds(o, GRP)], wgv)

                        def bb(b, _4):
                            _scale_scatter_batch(msg_h, acc, sgv, tgv, wgv,
                                                 rowsv, sem, b)
                            return 0

                        lax.fori_loop(0, GRP // 128, bb, 0)
                        return 0

                    lax.fori_loop(0, ng, gb, 0)
                    return 0

                lax.fori_loop(0, 2, tb, 0)

            plsc.subcore_barrier()

            @pl.when(own)
            def _wb():
                pltpu.sync_copy(acc.at[pl.ds(sid * 512, 512)],
                                out_h.at[pl.ds(c * CH + sid * 512, 512)])

            return 0

        lax.fori_loop(0, NCHUNK, chunk_body, 0)

    return k(msg, bs, bt, bw, cnt, zeros64)


def _edge_pass_t2(msg, es, w, zeros64):
    @functools.partial(
        pl.kernel,
        out_type=jax.ShapeDtypeStruct((NC, TP, D), jnp.float32),
        mesh=_mesh(),
        compiler_params=_sc_params(),
        scratch_types=[pltpu.VMEM_SHARED((TP, D), jnp.float32),
                       pltpu.VMEM((PW,), jnp.int32),
                       pltpu.VMEM((PW,), jnp.int32),
                       pltpu.VMEM((PW,), jnp.float32),
                       pltpu.VMEM((128, D), jnp.float32),
                       pltpu.VMEM((64, D), jnp.float32),
                       pltpu.SemaphoreType.DMA],
    )
    def k(msg_h, src_h, tgt_h, w_h, z_h, out_h, acc, sgv, tgv, wgv, rowsv,
          zbuf, sem):
        cid = lax.axis_index("c")
        sid = lax.axis_index("s")
        wid = _wid()
        base = wid * PW
        pltpu.sync_copy(z_h, zbuf)
        pltpu.sync_copy(zbuf, acc.at[pl.ds(sid * 64, 64)])
        plsc.subcore_barrier()
        pltpu.sync_copy(src_h.at[pl.ds(base, PW)], sgv)
        pltpu.sync_copy(tgt_h.at[pl.ds(base, PW)], tgv)
        pltpu.sync_copy(w_h.at[pl.ds(base, PW)], wgv)

        def bb(b, _):
            _scale_scatter_batch(msg_h, acc, sgv, tgv, wgv, rowsv, sem, b)
            return 0

        lax.fori_loop(0, PW // 128, bb, 0)
        plsc.subcore_barrier()
        pltpu.sync_copy(acc.at[pl.ds(sid * 64, 64)],
                        out_h.at[cid, pl.ds(sid * 64, 64)])

    return k(msg, es[0], es[1], w, zeros64)


def kernel(cl_idx, cc_feat, al_idx, ac_feat, test_idx, es0, es1, es2,
           w0, w1, w2, enc_cl_tab, enc_cc_W, enc_cc_b, enc_al_tab, enc_ac_W,
           enc_ac_b, emb_test_tab, Wm, bm, Wd, bd):
    f32 = jnp.float32

    def pad_edges(es, w):
        esp = jnp.zeros((2, EP), es.dtype).at[:, :E].set(es)
        wp = jnp.zeros((EP,), f32).at[:E].set(w)
        return (esp[0], esp[1]), wp

    es0p, w0p = pad_edges(es0, w0)
    es1p, w1p = pad_edges(es1, w1)
    es2p, w2p = pad_edges(es2, w2)
    cl_p = jnp.zeros((NE,), cl_idx.dtype).at[:N_NODE].set(cl_idx)
    al_p = jnp.zeros((NE,), al_idx.dtype).at[:N_NODE].set(al_idx)
    zeros64 = jnp.zeros((64, D), f32)

    g0, g1 = _emb_gather2(enc_cl_tab, cl_p, enc_al_tab, al_p)
    x0 = _enc(g0, cc_feat, enc_cc_W, enc_cc_b)
    x1 = _enc(g1, ac_feat, enc_ac_W, enc_ac_b)

    bs0, bt0, bw0, cnt0 = _bucket(es0p, w0p)
    bs1, bt1, bw1, cnt1 = _bucket(es1p, w1p)

    in_relu = False
    for i in range(4):
        m0 = _mm(x0, Wm[i, 0], bm[i, 0], in_relu, True)
        m1 = _mm(x1, Wm[i, 1], bm[i, 1], in_relu, True)
        x1 = _edge_pass(m0, bs0, bt0, bw0, cnt0, zeros64)
        x0 = _edge_pass(m1, bs1, bt1, bw1, cnt1, zeros64)
        in_relu = True

    m2 = _mm(x0, Wm[4, 2], bm[4, 2], True, True)
    t2 = _edge_pass_t2(m2, es2p, w2p, zeros64)

    Wdp = jnp.zeros((D, D), f32).at[:, :3].set(Wd)
    bdp = jnp.full((D,), -1e30, f32).at[:3].set(bd)
    zp, pp = _dec(t2[0], t2[1], Wdp, bdp)
    return zp[:N_TEST, :3], pp[:N_TEST, :3]
